# baseline (speedup 1.0000x reference)
import functools

import jax
import jax.numpy as jnp
from jax import lax
from jax.experimental import pallas as pl
from jax.experimental.pallas import tpu as pltpu
from jax.experimental.pallas import tpu_sc as plsc

N = 10000
E = 320000
D = 128
G = 64
NC = 2
NS = 16
CB = 128
CHUNKS = 80
EPAD = NC * NS * CHUNKS * CB
NPAD = 10240
RPT = NPAD // NS
DEGP = 10240
NB = 16
R = NPAD // NB

@functools.cache
def _mesh():
    return plsc.VectorSubcoreMesh(core_axis_name="c", subcore_axis_name="s",
                                  num_cores=NC, num_subcores=NS)


def _sc_deg_body(dst_hbm, degp_hbm, dst_v, deg_v):
    w = lax.axis_index("c") * NS + lax.axis_index("s")
    pltpu.sync_copy(dst_hbm.at[w], dst_v)

    zeros16 = jnp.zeros((16,), jnp.float32)
    ones16 = jnp.ones((16,), jnp.float32)

    def zbody(i, _):
        deg_v[pl.ds(i * 16, 16)] = zeros16
        return 0
    lax.fori_loop(0, DEGP // 16, zbody, 0)

    def hbody(i, _):
        idx = dst_v[pl.ds(i * 16, 16)]
        plsc.addupdate_scatter(deg_v, [idx], ones16)
        return 0
    lax.fori_loop(0, (CHUNKS * CB) // 16, hbody, 0)

    pltpu.sync_copy(deg_v, degp_hbm.at[w])


def _sc_deg(dst_flat):
    return pl.kernel(
        _sc_deg_body,
        out_type=jax.ShapeDtypeStruct((NC * NS, DEGP), jnp.float32),
        mesh=_mesh(),
        scratch_types=[
            pltpu.VMEM((CHUNKS * CB,), jnp.int32),
            pltpu.VMEM((DEGP,), jnp.float32),
        ],
        compiler_params=pltpu.CompilerParams(needs_layout_passes=False),
    )(dst_flat)


WIN = 40
NSPLIT = 1
C0 = 40
C1 = 2 * CHUNKS - C0
NCHUNK_TOT = NS * (C0 + C1)


def _sc_agg_body(hs_hbm, src_hbm, dst_hbm, zrow_hbm, out_hbm,
                 src_v, dst_v, buf_a, buf_b, acc_sh,
                 gsem_a, gsem_b, ssem_a, ssem_b):
    c = lax.axis_index("c")
    s = lax.axis_index("s")

    pltpu.sync_copy(zrow_hbm, acc_sh.at[pl.ds(s * RPT, RPT)])
    plsc.subcore_barrier()

    QR = CB // NSPLIT

    def gather(j, buf, gsem):
        for q in range(NSPLIT):
            pltpu.async_copy(hs_hbm.at[src_v.at[j, pl.ds(q * QR, QR)]],
                             buf.at[pl.ds(q * QR, QR)], gsem)

    def gather_wait(j, buf, gsem):
        for q in range(NSPLIT):
            pltpu.make_async_copy(hs_hbm.at[src_v.at[j, pl.ds(q * QR, QR)]],
                                  buf.at[pl.ds(q * QR, QR)], gsem).wait()

    def scat(j, buf, ssem):
        return pltpu.async_copy(buf, acc_sh.at[dst_v.at[j]], ssem, add=True)

    def scat_wait(j, buf, ssem):
        pltpu.make_async_copy(buf, acc_sh.at[dst_v.at[j]], ssem).wait()

    base = jnp.where(c == 0, s * C0, NS * C0 + s * C1)
    nwin = jnp.where(c == 0, C0 // WIN, C1 // WIN)

    def window(h, _):
        pltpu.sync_copy(src_hbm.at[pl.ds(base + h * WIN, WIN)], src_v)
        pltpu.sync_copy(dst_hbm.at[pl.ds(base + h * WIN, WIN)], dst_v)
        gather(0, buf_a, gsem_a)

        def body(k, _):
            j = 2 * k
            gather(j + 1, buf_b, gsem_b)
            gather_wait(j, buf_a, gsem_a)
            scat(j, buf_a, ssem_a)
            gather_wait(j + 1, buf_b, gsem_b)
            scat(j + 1, buf_b, ssem_b)
            scat_wait(j, buf_a, ssem_a)

            @pl.when(j + 2 < WIN)
            def _():
                gather(j + 2, buf_a, gsem_a)

            scat_wait(j + 1, buf_b, ssem_b)
            return 0

        lax.fori_loop(0, WIN // 2, body, 0)
        return 0

    lax.fori_loop(0, nwin, window, 0)

    plsc.subcore_barrier()
    pltpu.sync_copy(acc_sh.at[pl.ds(s * RPT, RPT)],
                    out_hbm.at[c, pl.ds(s * RPT, RPT)])


def _sc_agg(hs, src_p, dst_p, zrow):
    return pl.kernel(
        _sc_agg_body,
        out_type=jax.ShapeDtypeStruct((NC, NPAD, D), jnp.float32),
        mesh=_mesh(),
        scratch_types=[
            pltpu.VMEM((WIN, CB), jnp.int32),
            pltpu.VMEM((WIN, CB), jnp.int32),
            pltpu.VMEM((CB, D), jnp.float32),
            pltpu.VMEM((CB, D), jnp.float32),
            pltpu.VMEM_SHARED((NPAD, D), jnp.float32),
            pltpu.SemaphoreType.DMA,
            pltpu.SemaphoreType.DMA,
            pltpu.SemaphoreType.DMA,
            pltpu.SemaphoreType.DMA,
        ],
        compiler_params=pltpu.CompilerParams(needs_layout_passes=False),
    )(hs, src_p, dst_p, zrow)


def _dinv_from(degt):
    deg = jnp.sum(degt, axis=1, keepdims=True) + 1.0
    return lax.rsqrt(jnp.maximum(deg, 1.0))


def _tc_scale_mm_body(x_ref, w_ref, degt_ref, o_ref):
    dinv = _dinv_from(degt_ref[...])
    o_ref[...] = jnp.dot(x_ref[...], w_ref[...],
                         preferred_element_type=jnp.float32) * dinv


def _tc_layer_body(hs_ref, p0_ref, p1_ref, degt_ref, b_ref, w_ref, o_ref):
    i = pl.program_id(0)
    dinv = _dinv_from(degt_ref[...])
    h = dinv * (hs_ref[...] + p0_ref[...] + p1_ref[...]) + b_ref[...]
    h = jnp.maximum(h, 0.0)
    rows = i * R + lax.broadcasted_iota(jnp.int32, (R, 1), 0)
    h = jnp.where(rows < N, h, 0.0)
    o_ref[...] = jnp.dot(h, w_ref[...],
                         preferred_element_type=jnp.float32) * dinv


def _tc_pool_body(hs_ref, p0_ref, p1_ref, degt_ref, b_ref, batch_ref,
                  wl_ref, bl_ref, o_ref, sums, counts):
    i = pl.program_id(0)

    @pl.when(i == 0)
    def _():
        sums[...] = jnp.zeros_like(sums)
        counts[...] = jnp.zeros_like(counts)

    dinv = _dinv_from(degt_ref[...])
    h = dinv * (hs_ref[...] + p0_ref[...] + p1_ref[...]) + b_ref[...]
    h = jnp.maximum(h, 0.0)
    bb = batch_ref[...].reshape(1, R)
    gids = lax.broadcasted_iota(jnp.int32, (G, R), 0)
    mask = (gids == bb).astype(jnp.float32)
    sums[...] += jnp.dot(mask, h, preferred_element_type=jnp.float32)
    counts[...] += jnp.sum(mask, axis=1, keepdims=True)

    @pl.when(i == NB - 1)
    def _():
        g = sums[...] / jnp.maximum(counts[...], 1.0)
        o_ref[...] = jnp.dot(g, wl_ref[...],
                             preferred_element_type=jnp.float32) + bl_ref[...]


def _row_spec(i_map=lambda i: (i, 0)):
    return pl.BlockSpec((R, D), i_map)


_full_w = pl.BlockSpec((D, D), lambda i: (0, 0))
_degt_spec = pl.BlockSpec((R, 32), lambda i: (i, 0))
_bias_spec = pl.BlockSpec((1, D), lambda i: (0, 0))


def _tc_scale_mm(x, w, degt):
    return pl.pallas_call(
        _tc_scale_mm_body,
        grid=(NB,),
        in_specs=[_row_spec(), _full_w, _degt_spec],
        out_specs=_row_spec(),
        out_shape=jax.ShapeDtypeStruct((NPAD, D), jnp.float32),
        compiler_params=pltpu.CompilerParams(
            dimension_semantics=("arbitrary",)),
    )(x, w, degt)


def _tc_layer(hs, p0, p1, degt, b, w):
    return pl.pallas_call(
        _tc_layer_body,
        grid=(NB,),
        in_specs=[_row_spec(), _row_spec(), _row_spec(), _degt_spec,
                  _bias_spec, _full_w],
        out_specs=_row_spec(),
        out_shape=jax.ShapeDtypeStruct((NPAD, D), jnp.float32),
        input_output_aliases={0: 0},
        compiler_params=pltpu.CompilerParams(
            dimension_semantics=("arbitrary",)),
    )(hs, p0, p1, degt, b, w)


def _tc_pool(hs, p0, p1, degt, b, batch3, wl, bl):
    return pl.pallas_call(
        _tc_pool_body,
        grid=(NB,),
        in_specs=[_row_spec(), _row_spec(), _row_spec(), _degt_spec,
                  _bias_spec, pl.BlockSpec((1, 1, R), lambda i: (i, 0, 0)),
                  _full_w, _bias_spec],
        out_specs=pl.BlockSpec((G, D), lambda i: (0, 0)),
        out_shape=jax.ShapeDtypeStruct((G, D), jnp.float32),
        scratch_shapes=[pltpu.VMEM((G, D), jnp.float32),
                        pltpu.VMEM((G, 1), jnp.float32)],
        compiler_params=pltpu.CompilerParams(
            dimension_semantics=("arbitrary",)),
    )(hs, p0, p1, degt, b, batch3, wl, bl)


@jax.jit
def _run(x, edge_index, batch, W1, b1, W2, b2, Wl, bl):
    src = edge_index[0].astype(jnp.int32)
    dst = edge_index[1].astype(jnp.int32)
    padfill = jnp.full((EPAD - E,), N, jnp.int32)
    src_p = jnp.concatenate([src, padfill]).reshape(NCHUNK_TOT, CB)
    dst_p = jnp.concatenate([dst, padfill])
    dst_flat = dst_p.reshape(NC * NS, CHUNKS * CB)
    dst_p = dst_p.reshape(NCHUNK_TOT, CB)

    degp = _sc_deg(dst_flat)
    degt = degp.T[:NPAD]

    x_pad = jnp.zeros((NPAD, D), jnp.float32).at[:N].set(x)
    zrow = jnp.zeros((RPT, D), jnp.float32)
    batch3 = (jnp.concatenate([batch.astype(jnp.int32),
                               jnp.full((NPAD - N,), G, jnp.int32)])
              .reshape(NB, 1, R))
    wl_pad = jnp.zeros((D, D), jnp.float32).at[:, :2].set(Wl)
    bl_pad = jnp.zeros((1, D), jnp.float32).at[0, :2].set(bl)

    hs1 = _tc_scale_mm(x_pad, W1, degt)
    parts1 = _sc_agg(hs1, src_p, dst_p, zrow)
    hs2 = _tc_layer(hs1, parts1[0], parts1[1], degt,
                    b1.reshape(1, D), W2)
    parts2 = _sc_agg(hs2, src_p, dst_p, zrow)
    out = _tc_pool(hs2, parts2[0], parts2[1], degt, b2.reshape(1, D),
                   batch3, wl_pad, bl_pad)
    return out[:, :2]


def kernel(x, edge_index, batch, W1, b1, W2, b2, Wl, bl):
    return _run(x, edge_index, batch, W1, b1, W2, b2, Wl, bl)


# speedup vs baseline: 1.1587x; 1.1587x over previous
import functools

import jax
import jax.numpy as jnp
from jax import lax
from jax.experimental import pallas as pl
from jax.experimental.pallas import tpu as pltpu
from jax.experimental.pallas import tpu_sc as plsc

N = 10000
E = 320000
D = 128
G = 64
NC = 2
NS = 16
CB = 128
CHUNKS = 80
EPAD = NC * NS * CHUNKS * CB
NPAD = 10240
RPT = NPAD // NS
DEGP = 10240
NB = 16
R = NPAD // NB

@functools.cache
def _mesh():
    return plsc.VectorSubcoreMesh(core_axis_name="c", subcore_axis_name="s",
                                  num_cores=NC, num_subcores=NS)


def _sc_deg_body(dst_hbm, degp_hbm, dst_v, deg_v):
    w = lax.axis_index("c") * NS + lax.axis_index("s")
    pltpu.sync_copy(dst_hbm.at[w], dst_v)

    zeros16 = jnp.zeros((16,), jnp.float32)
    ones16 = jnp.ones((16,), jnp.float32)

    def zbody(i, _):
        deg_v[pl.ds(i * 16, 16)] = zeros16
        return 0
    lax.fori_loop(0, DEGP // 16, zbody, 0)

    def hbody(i, _):
        idx = dst_v[pl.ds(i * 16, 16)]
        plsc.addupdate_scatter(deg_v, [idx], ones16)
        return 0
    lax.fori_loop(0, (CHUNKS * CB) // 16, hbody, 0)

    pltpu.sync_copy(deg_v, degp_hbm.at[w])


def _sc_deg(dst_flat):
    return pl.kernel(
        _sc_deg_body,
        out_type=jax.ShapeDtypeStruct((NC * NS, DEGP), jnp.float32),
        mesh=_mesh(),
        scratch_types=[
            pltpu.VMEM((CHUNKS * CB,), jnp.int32),
            pltpu.VMEM((DEGP,), jnp.float32),
        ],
        compiler_params=pltpu.CompilerParams(needs_layout_passes=False),
    )(dst_flat)


WIN = 40
NSPLIT = 1
C0 = 120
C1 = 2 * CHUNKS - C0
NCHUNK_TOT = NS * (C0 + C1)


def _sc_agg_body(hs_hbm, src_hbm, dst_hbm, zrow_hbm, out_hbm,
                 src_v, dst_v, buf_a, buf_b, acc_sh,
                 gsem_a, gsem_b, ssem_a, ssem_b):
    c = lax.axis_index("c")
    s = lax.axis_index("s")

    pltpu.sync_copy(zrow_hbm, acc_sh.at[pl.ds(s * RPT, RPT)])
    plsc.subcore_barrier()

    QR = CB // NSPLIT

    def gather(j, buf, gsem):
        for q in range(NSPLIT):
            pltpu.async_copy(hs_hbm.at[src_v.at[j, pl.ds(q * QR, QR)]],
                             buf.at[pl.ds(q * QR, QR)], gsem)

    def gather_wait(j, buf, gsem):
        for q in range(NSPLIT):
            pltpu.make_async_copy(hs_hbm.at[src_v.at[j, pl.ds(q * QR, QR)]],
                                  buf.at[pl.ds(q * QR, QR)], gsem).wait()

    def scat(j, buf, ssem):
        return pltpu.async_copy(buf, acc_sh.at[dst_v.at[j]], ssem, add=True)

    def scat_wait(j, buf, ssem):
        pltpu.make_async_copy(buf, acc_sh.at[dst_v.at[j]], ssem).wait()

    base = jnp.where(c == 0, s * C0, NS * C0 + s * C1)
    nwin = jnp.where(c == 0, C0 // WIN, C1 // WIN)

    def window(h, _):
        pltpu.sync_copy(src_hbm.at[pl.ds(base + h * WIN, WIN)], src_v)
        pltpu.sync_copy(dst_hbm.at[pl.ds(base + h * WIN, WIN)], dst_v)
        gather(0, buf_a, gsem_a)

        def body(k, _):
            j = 2 * k
            gather(j + 1, buf_b, gsem_b)
            gather_wait(j, buf_a, gsem_a)
            scat(j, buf_a, ssem_a)
            gather_wait(j + 1, buf_b, gsem_b)
            scat(j + 1, buf_b, ssem_b)
            scat_wait(j, buf_a, ssem_a)

            @pl.when(j + 2 < WIN)
            def _():
                gather(j + 2, buf_a, gsem_a)

            scat_wait(j + 1, buf_b, ssem_b)
            return 0

        lax.fori_loop(0, WIN // 2, body, 0)
        return 0

    lax.fori_loop(0, nwin, window, 0)

    plsc.subcore_barrier()
    pltpu.sync_copy(acc_sh.at[pl.ds(s * RPT, RPT)],
                    out_hbm.at[c, pl.ds(s * RPT, RPT)])


def _sc_agg(hs, src_p, dst_p, zrow):
    return pl.kernel(
        _sc_agg_body,
        out_type=jax.ShapeDtypeStruct((NC, NPAD, D), jnp.float32),
        mesh=_mesh(),
        scratch_types=[
            pltpu.VMEM((WIN, CB), jnp.int32),
            pltpu.VMEM((WIN, CB), jnp.int32),
            pltpu.VMEM((CB, D), jnp.float32),
            pltpu.VMEM((CB, D), jnp.float32),
            pltpu.VMEM_SHARED((NPAD, D), jnp.float32),
            pltpu.SemaphoreType.DMA,
            pltpu.SemaphoreType.DMA,
            pltpu.SemaphoreType.DMA,
            pltpu.SemaphoreType.DMA,
        ],
        compiler_params=pltpu.CompilerParams(needs_layout_passes=False),
    )(hs, src_p, dst_p, zrow)


def _dinv_from(degt):
    deg = jnp.sum(degt, axis=1, keepdims=True) + 1.0
    return lax.rsqrt(jnp.maximum(deg, 1.0))


def _tc_scale_mm_body(x_ref, w_ref, degt_ref, o_ref):
    dinv = _dinv_from(degt_ref[...])
    o_ref[...] = jnp.dot(x_ref[...], w_ref[...],
                         preferred_element_type=jnp.float32) * dinv


def _tc_layer_body(hs_ref, p0_ref, p1_ref, degt_ref, b_ref, w_ref, o_ref):
    i = pl.program_id(0)
    dinv = _dinv_from(degt_ref[...])
    h = dinv * (hs_ref[...] + p0_ref[...] + p1_ref[...]) + b_ref[...]
    h = jnp.maximum(h, 0.0)
    rows = i * R + lax.broadcasted_iota(jnp.int32, (R, 1), 0)
    h = jnp.where(rows < N, h, 0.0)
    o_ref[...] = jnp.dot(h, w_ref[...],
                         preferred_element_type=jnp.float32) * dinv


def _tc_pool_body(hs_ref, p0_ref, p1_ref, degt_ref, b_ref, batch_ref,
                  wl_ref, bl_ref, o_ref, sums, counts):
    i = pl.program_id(0)

    @pl.when(i == 0)
    def _():
        sums[...] = jnp.zeros_like(sums)
        counts[...] = jnp.zeros_like(counts)

    dinv = _dinv_from(degt_ref[...])
    h = dinv * (hs_ref[...] + p0_ref[...] + p1_ref[...]) + b_ref[...]
    h = jnp.maximum(h, 0.0)
    bb = batch_ref[...].reshape(1, R)
    gids = lax.broadcasted_iota(jnp.int32, (G, R), 0)
    mask = (gids == bb).astype(jnp.float32)
    sums[...] += jnp.dot(mask, h, preferred_element_type=jnp.float32)
    counts[...] += jnp.sum(mask, axis=1, keepdims=True)

    @pl.when(i == NB - 1)
    def _():
        g = sums[...] / jnp.maximum(counts[...], 1.0)
        o_ref[...] = jnp.dot(g, wl_ref[...],
                             preferred_element_type=jnp.float32) + bl_ref[...]


def _row_spec(i_map=lambda i: (i, 0)):
    return pl.BlockSpec((R, D), i_map)


_full_w = pl.BlockSpec((D, D), lambda i: (0, 0))
_degt_spec = pl.BlockSpec((R, 32), lambda i: (i, 0))
_bias_spec = pl.BlockSpec((1, D), lambda i: (0, 0))


def _tc_scale_mm(x, w, degt):
    return pl.pallas_call(
        _tc_scale_mm_body,
        grid=(NB,),
        in_specs=[_row_spec(), _full_w, _degt_spec],
        out_specs=_row_spec(),
        out_shape=jax.ShapeDtypeStruct((NPAD, D), jnp.float32),
        compiler_params=pltpu.CompilerParams(
            dimension_semantics=("arbitrary",)),
    )(x, w, degt)


def _tc_layer(hs, p0, p1, degt, b, w):
    return pl.pallas_call(
        _tc_layer_body,
        grid=(NB,),
        in_specs=[_row_spec(), _row_spec(), _row_spec(), _degt_spec,
                  _bias_spec, _full_w],
        out_specs=_row_spec(),
        out_shape=jax.ShapeDtypeStruct((NPAD, D), jnp.float32),
        input_output_aliases={0: 0},
        compiler_params=pltpu.CompilerParams(
            dimension_semantics=("arbitrary",)),
    )(hs, p0, p1, degt, b, w)


def _tc_pool(hs, p0, p1, degt, b, batch3, wl, bl):
    return pl.pallas_call(
        _tc_pool_body,
        grid=(NB,),
        in_specs=[_row_spec(), _row_spec(), _row_spec(), _degt_spec,
                  _bias_spec, pl.BlockSpec((1, 1, R), lambda i: (i, 0, 0)),
                  _full_w, _bias_spec],
        out_specs=pl.BlockSpec((G, D), lambda i: (0, 0)),
        out_shape=jax.ShapeDtypeStruct((G, D), jnp.float32),
        scratch_shapes=[pltpu.VMEM((G, D), jnp.float32),
                        pltpu.VMEM((G, 1), jnp.float32)],
        compiler_params=pltpu.CompilerParams(
            dimension_semantics=("arbitrary",)),
    )(hs, p0, p1, degt, b, batch3, wl, bl)


@jax.jit
def _run(x, edge_index, batch, W1, b1, W2, b2, Wl, bl):
    src = edge_index[0].astype(jnp.int32)
    dst = edge_index[1].astype(jnp.int32)
    padfill = jnp.full((EPAD - E,), N, jnp.int32)
    src_p = jnp.concatenate([src, padfill]).reshape(NCHUNK_TOT, CB)
    dst_p = jnp.concatenate([dst, padfill])
    dst_flat = dst_p.reshape(NC * NS, CHUNKS * CB)
    dst_p = dst_p.reshape(NCHUNK_TOT, CB)

    degp = _sc_deg(dst_flat)
    degt = degp.T[:NPAD]

    x_pad = jnp.zeros((NPAD, D), jnp.float32).at[:N].set(x)
    zrow = jnp.zeros((RPT, D), jnp.float32)
    batch3 = (jnp.concatenate([batch.astype(jnp.int32),
                               jnp.full((NPAD - N,), G, jnp.int32)])
              .reshape(NB, 1, R))
    wl_pad = jnp.zeros((D, D), jnp.float32).at[:, :2].set(Wl)
    bl_pad = jnp.zeros((1, D), jnp.float32).at[0, :2].set(bl)

    hs1 = _tc_scale_mm(x_pad, W1, degt)
    parts1 = _sc_agg(hs1, src_p, dst_p, zrow)
    hs2 = _tc_layer(hs1, parts1[0], parts1[1], degt,
                    b1.reshape(1, D), W2)
    parts2 = _sc_agg(hs2, src_p, dst_p, zrow)
    out = _tc_pool(hs2, parts2[0], parts2[1], degt, b2.reshape(1, D),
                   batch3, wl_pad, bl_pad)
    return out[:, :2]


def kernel(x, edge_index, batch, W1, b1, W2, b2, Wl, bl):
    return _run(x, edge_index, batch, W1, b1, W2, b2, Wl, bl)
